# baseline (speedup 1.0000x reference)
import jax
import jax.numpy as jnp
from jax.experimental import pallas as pl
from jax.experimental.pallas import tpu as pltpu


def _moe_block_kernel(xf_ref, xh_ref, gw_ref, gb_ref, ew_ref, out_ref, g_ref):
    e = pl.program_id(1)

    @pl.when(e == 0)
    def _init():
        logits = (
            jnp.dot(xf_ref[...], gw_ref[...], preferred_element_type=jnp.float32)
            + gb_ref[...]
        )
        m = jnp.max(logits, axis=-1, keepdims=True)
        p = jnp.exp(logits - m)
        g_ref[...] = p / jnp.sum(p, axis=-1, keepdims=True)
        out_ref[...] = jnp.zeros_like(out_ref)

    num_e = g_ref.shape[1]
    lane = jax.lax.broadcasted_iota(jnp.int32, (1, num_e), 1)
    g_col = jnp.sum(
        jnp.where(lane == e, g_ref[...], 0.0), axis=1, keepdims=True
    )
    y = jnp.dot(xh_ref[...], ew_ref[0], preferred_element_type=jnp.float32)
    out_ref[...] += g_col * y


def kernel(x, gate_W, gate_b, expert_W):
    B, S, D_IN = x.shape
    E, _, D_OUT = expert_W.shape
    x2 = x.reshape(B * S, D_IN)
    xh = x2.astype(jnp.bfloat16)
    ewh = expert_W.astype(jnp.bfloat16)
    gb2 = gate_b.reshape(1, E)

    S_BLK = 2048
    n_s = (B * S) // S_BLK

    out = pl.pallas_call(
        _moe_block_kernel,
        grid=(n_s, E),
        in_specs=[
            pl.BlockSpec((S_BLK, D_IN), lambda s, e: (s, 0)),
            pl.BlockSpec((S_BLK, D_IN), lambda s, e: (s, 0)),
            pl.BlockSpec((D_IN, E), lambda s, e: (0, 0)),
            pl.BlockSpec((1, E), lambda s, e: (0, 0)),
            pl.BlockSpec((1, D_IN, D_OUT), lambda s, e: (e, 0, 0)),
        ],
        out_specs=pl.BlockSpec((S_BLK, D_OUT), lambda s, e: (s, 0)),
        out_shape=jax.ShapeDtypeStruct((B * S, D_OUT), jnp.float32),
        scratch_shapes=[pltpu.VMEM((S_BLK, E), jnp.float32)],
        compiler_params=pltpu.CompilerParams(
            dimension_semantics=("parallel", "arbitrary"),
        ),
    )(x2, xh, gate_W, gb2, ewh)
    return out.reshape(B, S, D_OUT)


# speedup vs baseline: 1.4100x; 1.4100x over previous
import jax
import jax.numpy as jnp
from jax.experimental import pallas as pl
from jax.experimental.pallas import tpu as pltpu


def _moe_block_kernel(x_ref, gw_ref, gb_ref, ew_ref, out_ref, g_ref):
    e = pl.program_id(1)

    @pl.when(e == 0)
    def _init():
        logits = (
            jnp.dot(x_ref[...], gw_ref[...], preferred_element_type=jnp.float32)
            + gb_ref[...]
        )
        m = jnp.max(logits, axis=-1, keepdims=True)
        p = jnp.exp(logits - m)
        g_ref[...] = p / jnp.sum(p, axis=-1, keepdims=True)
        out_ref[...] = jnp.zeros_like(out_ref)

    num_e = g_ref.shape[1]
    lane = jax.lax.broadcasted_iota(jnp.int32, (1, num_e), 1)
    g_col = jnp.sum(
        jnp.where(lane == e, g_ref[...], 0.0), axis=1, keepdims=True
    )
    xh = x_ref[...].astype(jnp.bfloat16)
    wh = ew_ref[0].astype(jnp.bfloat16)
    y = jnp.dot(xh, wh, preferred_element_type=jnp.float32)
    out_ref[...] += g_col * y


def kernel(x, gate_W, gate_b, expert_W):
    B, S, D_IN = x.shape
    E, _, D_OUT = expert_W.shape
    x2 = x.reshape(B * S, D_IN)
    gb2 = gate_b.reshape(1, E)

    S_BLK = 2048
    n_s = (B * S) // S_BLK

    out = pl.pallas_call(
        _moe_block_kernel,
        grid=(n_s, E),
        in_specs=[
            pl.BlockSpec((S_BLK, D_IN), lambda s, e: (s, 0)),
            pl.BlockSpec((D_IN, E), lambda s, e: (0, 0)),
            pl.BlockSpec((1, E), lambda s, e: (0, 0)),
            pl.BlockSpec((1, D_IN, D_OUT), lambda s, e: (e, 0, 0)),
        ],
        out_specs=pl.BlockSpec((S_BLK, D_OUT), lambda s, e: (s, 0)),
        out_shape=jax.ShapeDtypeStruct((B * S, D_OUT), jnp.float32),
        scratch_shapes=[pltpu.VMEM((S_BLK, E), jnp.float32)],
        compiler_params=pltpu.CompilerParams(
            dimension_semantics=("parallel", "arbitrary"),
        ),
    )(x2, gate_W, gb2, expert_W)
    return out.reshape(B, S, D_OUT)


# speedup vs baseline: 1.4269x; 1.0120x over previous
import jax
import jax.numpy as jnp
from jax.experimental import pallas as pl
from jax.experimental.pallas import tpu as pltpu


def _moe_block_kernel(x_ref, gw_ref, gb_ref, ew_ref, out_ref, g_ref):
    eb = pl.program_id(1)
    EB = ew_ref.shape[0]

    @pl.when(eb == 0)
    def _init():
        logits = (
            jnp.dot(x_ref[...], gw_ref[...], preferred_element_type=jnp.float32)
            + gb_ref[...]
        )
        m = jnp.max(logits, axis=-1, keepdims=True)
        p = jnp.exp(logits - m)
        g_ref[...] = p / jnp.sum(p, axis=-1, keepdims=True)
        out_ref[...] = jnp.zeros_like(out_ref)

    num_e = g_ref.shape[1]
    lane = jax.lax.broadcasted_iota(jnp.int32, (1, num_e), 1)
    g_all = g_ref[...]
    acc = out_ref[...]
    for k in range(EB):
        e = eb * EB + k
        g_col = jnp.sum(jnp.where(lane == e, g_all, 0.0), axis=1, keepdims=True)
        y = jnp.dot(x_ref[...], ew_ref[k], preferred_element_type=jnp.float32)
        acc = acc + g_col * y
    out_ref[...] = acc


def kernel(x, gate_W, gate_b, expert_W):
    B, S, D_IN = x.shape
    E, _, D_OUT = expert_W.shape
    x2 = x.reshape(B * S, D_IN)
    gb2 = gate_b.reshape(1, E)

    S_BLK = 2048
    EB = 4
    n_s = (B * S) // S_BLK

    out = pl.pallas_call(
        _moe_block_kernel,
        grid=(n_s, E // EB),
        in_specs=[
            pl.BlockSpec((S_BLK, D_IN), lambda s, e: (s, 0)),
            pl.BlockSpec((D_IN, E), lambda s, e: (0, 0)),
            pl.BlockSpec((1, E), lambda s, e: (0, 0)),
            pl.BlockSpec((EB, D_IN, D_OUT), lambda s, e: (e, 0, 0)),
        ],
        out_specs=pl.BlockSpec((S_BLK, D_OUT), lambda s, e: (s, 0)),
        out_shape=jax.ShapeDtypeStruct((B * S, D_OUT), jnp.float32),
        scratch_shapes=[pltpu.VMEM((S_BLK, E), jnp.float32)],
        compiler_params=pltpu.CompilerParams(
            dimension_semantics=("parallel", "arbitrary"),
        ),
    )(x2, gate_W, gb2, expert_W)
    return out.reshape(B, S, D_OUT)
